# baseline (speedup 1.0000x reference)
import functools

import jax
import jax.numpy as jnp
from jax import lax
from jax.experimental import pallas as pl
from jax.experimental.pallas import tpu as pltpu
from jax.experimental.pallas import tpu_sc as plsc

_DIM = 32
_BATCH = 16384
_NW = 32
_PAIRS_PER_W = _BATCH // _NW
_ROWS_PER_W = 2 * _PAIRS_PER_W
_CHUNK = 128
_NCHUNK = _ROWS_PER_W // _CHUNK
_GROUPS = _PAIRS_PER_W // 16
_NEWTON_ITERS = 6


def _sc_body(idx_hbm, table_hbm, out_hbm, idx_v, rows_v, out_v, sem):
    wid = lax.axis_index("s") * 2 + lax.axis_index("c")

    pltpu.sync_copy(idx_hbm.at[pl.ds(wid * _NCHUNK, _NCHUNK)], idx_v)

    copies = [
        pltpu.async_copy(
            table_hbm.at[idx_v.at[j]],
            rows_v.at[pl.ds(j * _CHUNK, _CHUNK)],
            sem,
        )
        for j in range(_NCHUNK)
    ]
    for c in copies:
        c.wait()

    lanes = jnp.arange(16, dtype=jnp.int32)

    def group_body(g, carry):
        row_u = g * 32 + 2 * lanes
        row_v = row_u + 1
        uu = jnp.zeros((16,), jnp.float32)
        vv = jnp.zeros((16,), jnp.float32)
        uv = jnp.zeros((16,), jnp.float32)
        for d in range(_DIM):
            col = jnp.full((16,), d, dtype=jnp.int32)
            u = plsc.load_gather(rows_v, [row_u, col])
            v = plsc.load_gather(rows_v, [row_v, col])
            uu = uu + u * u
            vv = vv + v * v
            uv = uv + u * v
        x = (uu + 1.0) * (vv + 1.0)
        y = 0.5 * (x + 1.0)
        for _ in range(_NEWTON_ITERS):
            y = 0.5 * (y + x / y)
        dist = 2.0 * y - 2.0 * uv + (-2.0 + 1e-5)
        out_v[pl.ds(g * 16, 16)] = dist
        return carry

    lax.fori_loop(0, _GROUPS, group_body, 0)

    pltpu.sync_copy(out_v, out_hbm.at[pl.ds(wid * _PAIRS_PER_W, _PAIRS_PER_W)])


@functools.partial(jax.jit, static_argnums=())
def kernel(idxs, table):
    idx_flat = idxs.reshape(_NW * _NCHUNK, _CHUNK)
    run = pl.kernel(
        _sc_body,
        out_type=jax.ShapeDtypeStruct((_BATCH,), jnp.float32),
        mesh=plsc.VectorSubcoreMesh(core_axis_name="c", subcore_axis_name="s"),
        scratch_types=[
            pltpu.VMEM((_NCHUNK, _CHUNK), jnp.int32),
            pltpu.VMEM((_ROWS_PER_W, _DIM), jnp.float32),
            pltpu.VMEM((_PAIRS_PER_W,), jnp.float32),
            pltpu.SemaphoreType.DMA,
        ],
        compiler_params=pltpu.CompilerParams(
            needs_layout_passes=False, use_tc_tiling_on_sc=False
        ),
    )
    return run(idx_flat, table)
